# baseline (speedup 1.0000x reference)
import functools

import jax
import jax.numpy as jnp
from jax import lax
from jax.experimental import pallas as pl
from jax.experimental.pallas import tpu as pltpu
from jax.experimental.pallas import tpu_sc as plsc

N = 10000
NP = 10240
D = 128
E = 320000
NC = 2
NS = 16
NW = NC * NS
B = 128
CH = 80
PASSES = 2
CHP = CH // PASSES
EP = NW * CH * B
RPT = NP // NS
_BETA = 0.5
_CMIX = 1.0

@functools.cache
def _mesh():
    return plsc.VectorSubcoreMesh(core_axis_name="c", subcore_axis_name="s",
                                  num_cores=NC, num_subcores=NS)


def _deg_body(eb, deg1d, ev, onesb, zb, acc, dsem):
    cid = lax.axis_index("c")
    sid = lax.axis_index("s")
    wid = cid * NS + sid
    one = jnp.ones((16,), jnp.float32)
    for r in range(B // 16):
        onesb[pl.ds(r * 16, 16)] = one
    z = jnp.zeros((16,), jnp.float32)
    for r in range(RPT // 16):
        zb[pl.ds(r * 16, 16)] = z
    pltpu.sync_copy(zb, acc.at[pl.ds(sid * RPT, RPT)])
    pltpu.sync_copy(eb.at[pl.ds(wid * 2 * CH, 2 * CH)], ev)
    plsc.subcore_barrier()

    def body(j, _):
        pltpu.async_copy(onesb, acc.at[ev.at[2 * j + 1]], dsem, add=True)
        return 0

    lax.fori_loop(0, CH, body, 0)

    def drain(j, _):
        pltpu.make_async_copy(onesb, acc.at[ev.at[2 * j + 1]], dsem).wait()
        return 0

    lax.fori_loop(0, CH, drain, 0)
    plsc.subcore_barrier()
    pltpu.sync_copy(acc.at[pl.ds(sid * RPT, RPT)],
                    deg1d.at[pl.ds(cid * NP + sid * RPT, RPT)])


def _deg_call(eb):
    return pl.kernel(
        _deg_body,
        out_type=jax.ShapeDtypeStruct((NC * NP,), jnp.float32),
        mesh=_mesh(),
        scratch_types=[
            pltpu.VMEM((2 * CH, B), jnp.int32),
            pltpu.VMEM((B,), jnp.float32),
            pltpu.VMEM((RPT,), jnp.float32),
            pltpu.VMEM_SHARED((NP,), jnp.float32),
            pltpu.SemaphoreType.DMA,
        ],
    )(eb)


def _scat_body(yp, eb, outp, ev, rows0, rows1, acc, gsem):
    cid = lax.axis_index("c")
    sid = lax.axis_index("s")
    wid = cid * NS + sid
    rs = pl.ds(sid * RPT, RPT)

    @pl.when(cid == 0)
    def _():
        pltpu.sync_copy(yp.at[rs], acc.at[rs])

    @pl.when(cid != 0)
    def _():
        z = jnp.zeros((16,), jnp.float32)
        for r in range(B):
            for c in range(D // 16):
                rows0[r, pl.ds(c * 16, 16)] = z
        for r in range(RPT // B):
            pltpu.sync_copy(rows0, acc.at[pl.ds(sid * RPT + r * B, B)])

    plsc.subcore_barrier()

    def passloop(p, _):
        pltpu.sync_copy(eb.at[pl.ds((wid * PASSES + p) * 2 * CHP, 2 * CHP)], ev)
        pltpu.async_copy(yp.at[ev.at[0]], rows0, gsem)

        def body(i, _):
            j = 2 * i
            pltpu.make_async_copy(yp.at[ev.at[2 * j]], rows0, gsem).wait()
            pltpu.async_copy(yp.at[ev.at[2 * j + 2]], rows1, gsem)
            pltpu.sync_copy(rows0, acc.at[ev.at[2 * j + 1]], add=True)
            pltpu.make_async_copy(yp.at[ev.at[2 * j + 2]], rows1, gsem).wait()

            @pl.when(j + 2 < CHP)
            def _():
                pltpu.async_copy(yp.at[ev.at[2 * j + 4]], rows0, gsem)

            pltpu.sync_copy(rows1, acc.at[ev.at[2 * j + 3]], add=True)
            return 0

        lax.fori_loop(0, CHP // 2, body, 0)
        return 0

    lax.fori_loop(0, PASSES, passloop, 0)
    plsc.subcore_barrier()
    pltpu.sync_copy(acc.at[rs], outp.at[cid, rs])


def _scat_call(yp, eb):
    return pl.kernel(
        _scat_body,
        out_type=jax.ShapeDtypeStruct((NC, NP, D), jnp.float32),
        mesh=_mesh(),
        scratch_types=[
            pltpu.VMEM((2 * CHP, B), jnp.int32),
            pltpu.VMEM((B, D), jnp.float32),
            pltpu.VMEM((B, D), jnp.float32),
            pltpu.VMEM_SHARED((NP, D), jnp.float32),
            pltpu.SemaphoreType.DMA,
        ],
    )(yp, eb)


def _diag(v):
    eq = (lax.broadcasted_iota(jnp.int32, (D, D), 0)
          == lax.broadcasted_iota(jnp.int32, (D, D), 1))
    return jnp.where(eq, jnp.broadcast_to(v[None, :], (D, D)), 0.0)


def _rowscale(v_packed, m):
    out = []
    for k in range(_RB // D):
        dm = _diag(v_packed[k])
        out.append(jnp.dot(dm, m[k * D:(k + 1) * D],
                           preferred_element_type=jnp.float32,
                           precision=lax.Precision.HIGHEST))
    return jnp.concatenate(out, axis=0)


def _t1_body(x_ref, w_ref, degp_ref, y_ref, dinvp_ref):
    deg = degp_ref[0] + degp_ref[1] + 1.0
    dv = lax.rsqrt(deg)
    dinvp_ref[...] = dv
    xw = jnp.dot(x_ref[...], w_ref[...],
                 preferred_element_type=jnp.float32,
                 precision=lax.Precision.HIGHEST)
    y_ref[...] = _rowscale(dv, xw)


def _mix(z, g_ref):
    ga = g_ref[0]
    gb = g_ref[1]
    gc = g_ref[2]
    gd = g_ref[3]
    gr = jnp.where(z < 0, ga * z, z)
    gr = jnp.where((z >= 0) & (z < gc), gb * z, gr)
    gr = jnp.where(z >= gc, gd * z, gr)
    return _BETA * z + (_CMIX - _BETA) * gr


def _t2_body(p_ref, dinvp_ref, b_ref, g_ref, w_ref, y2_ref):
    dv = dinvp_ref[...]
    z = _rowscale(dv, p_ref[0] + p_ref[1]) + b_ref[...][None, :]
    h = _mix(z, g_ref)
    xw2 = jnp.dot(h, w_ref[...],
                  preferred_element_type=jnp.float32,
                  precision=lax.Precision.HIGHEST)
    y2_ref[...] = _rowscale(dv, xw2)


def _t3_body(p_ref, dinvp_ref, b_ref, g_ref, o_ref):
    z = _rowscale(dinvp_ref[...], p_ref[0] + p_ref[1]) + b_ref[...][None, :]
    o_ref[...] = _mix(z, g_ref)


_RB = 1024
_GRID = NP // _RB


def _row_specs():
    sp_p = pl.BlockSpec((NC, _RB, D), lambda i: (0, i, 0))
    sp_r = pl.BlockSpec((_RB, D), lambda i: (i, 0))
    sp_c = pl.BlockSpec((_RB // D, D), lambda i: (i, 0))
    sp_b = pl.BlockSpec((D,), lambda i: (0,))
    sp_g = pl.BlockSpec((4,), lambda i: (0,))
    sp_w = pl.BlockSpec((D, D), lambda i: (0, 0))
    return sp_p, sp_r, sp_c, sp_b, sp_g, sp_w


def kernel(x, edge_index, W1, b1, g1, W2, b2, g2):
    real = edge_index.reshape(2, E // B, B).transpose(1, 0, 2).reshape(2 * E // B, B)
    npad = EP - E
    padblk = (N + (jnp.arange(2 * npad, dtype=jnp.int32) % (NP - N))
              ).reshape(2 * npad // B, B)
    eb = jnp.concatenate([real, padblk], axis=0)

    xp = jnp.pad(x, ((0, NP - N), (0, 0)))

    degp = _deg_call(eb).reshape(NC, NP // D, D)

    sp_p, sp_r, sp_c, sp_b, sp_g, sp_w = _row_specs()
    y1, dinvp = pl.pallas_call(
        _t1_body,
        grid=(_GRID,),
        in_specs=[sp_r, sp_w,
                  pl.BlockSpec((NC, _RB // D, D), lambda i: (0, i, 0))],
        out_specs=(sp_r, sp_c),
        out_shape=(jax.ShapeDtypeStruct((NP, D), jnp.float32),
                   jax.ShapeDtypeStruct((NP // D, D), jnp.float32)),
    )(xp, W1, degp)

    p1 = _scat_call(y1, eb)

    y2 = pl.pallas_call(
        _t2_body,
        grid=(_GRID,),
        in_specs=[sp_p, sp_c, sp_b, sp_g, sp_w],
        out_specs=sp_r,
        out_shape=jax.ShapeDtypeStruct((NP, D), jnp.float32),
    )(p1, dinvp, b1, g1, W2)

    p2 = _scat_call(y2, eb)

    return pl.pallas_call(
        _t3_body,
        grid=(_GRID,),
        in_specs=[sp_p, sp_c, sp_b, sp_g],
        out_specs=sp_r,
        out_shape=jax.ShapeDtypeStruct((N, D), jnp.float32),
    )(p2, dinvp, b2, g2)


# speedup vs baseline: 1.0414x; 1.0414x over previous
import functools

import jax
import jax.numpy as jnp
from jax import lax
from jax.experimental import pallas as pl
from jax.experimental.pallas import tpu as pltpu
from jax.experimental.pallas import tpu_sc as plsc

N = 10000
NP = 10240
D = 128
E = 320000
NC = 2
NS = 16
NW = NC * NS
B = 128
CH = 80
PASSES = 2
CHP = CH // PASSES
EP = NW * CH * B
RPT = NP // NS
_BETA = 0.5
_CMIX = 1.0

@functools.cache
def _mesh():
    return plsc.VectorSubcoreMesh(core_axis_name="c", subcore_axis_name="s",
                                  num_cores=NC, num_subcores=NS)


def _deg_body(eb, deg1d, ev, onesb, zb, acc, dsem):
    cid = lax.axis_index("c")
    sid = lax.axis_index("s")
    wid = cid * NS + sid
    one = jnp.ones((16,), jnp.float32)
    for r in range(B // 16):
        onesb[pl.ds(r * 16, 16)] = one
    z = jnp.zeros((16,), jnp.float32)
    for r in range(RPT // 16):
        zb[pl.ds(r * 16, 16)] = z
    pltpu.sync_copy(zb, acc.at[pl.ds(sid * RPT, RPT)])
    pltpu.sync_copy(eb.at[pl.ds(wid * 2 * CH, 2 * CH)], ev)
    plsc.subcore_barrier()

    def body(j, _):
        pltpu.async_copy(onesb, acc.at[ev.at[2 * j + 1]], dsem, add=True)
        return 0

    lax.fori_loop(0, CH, body, 0)

    def drain(j, _):
        pltpu.make_async_copy(onesb, acc.at[ev.at[2 * j + 1]], dsem).wait()
        return 0

    lax.fori_loop(0, CH, drain, 0)
    plsc.subcore_barrier()
    pltpu.sync_copy(acc.at[pl.ds(sid * RPT, RPT)],
                    deg1d.at[pl.ds(cid * NP + sid * RPT, RPT)])


def _deg_call(eb):
    return pl.kernel(
        _deg_body,
        out_type=jax.ShapeDtypeStruct((NC * NP,), jnp.float32),
        mesh=_mesh(),
        scratch_types=[
            pltpu.VMEM((2 * CH, B), jnp.int32),
            pltpu.VMEM((B,), jnp.float32),
            pltpu.VMEM((RPT,), jnp.float32),
            pltpu.VMEM_SHARED((NP,), jnp.float32),
            pltpu.SemaphoreType.DMA,
        ],
    )(eb)


def _scat_body(yp, eb, outp, ev, rows0, rows1, acc, gsem):
    cid = lax.axis_index("c")
    sid = lax.axis_index("s")
    wid = cid * NS + sid
    rs = pl.ds(sid * RPT, RPT)

    @pl.when(cid == 0)
    def _():
        pltpu.sync_copy(yp.at[rs], acc.at[rs])

    @pl.when(cid != 0)
    def _():
        z = jnp.zeros((16,), jnp.float32)
        for r in range(B):
            for c in range(D // 16):
                rows0[r, pl.ds(c * 16, 16)] = z
        for r in range(RPT // B):
            pltpu.sync_copy(rows0, acc.at[pl.ds(sid * RPT + r * B, B)])

    plsc.subcore_barrier()

    def passloop(p, _):
        pltpu.sync_copy(eb.at[pl.ds((wid * PASSES + p) * 2 * CHP, 2 * CHP)], ev)
        pltpu.async_copy(yp.at[ev.at[0]], rows0, gsem)

        def body(i, _):
            j = 2 * i
            pltpu.make_async_copy(yp.at[ev.at[2 * j]], rows0, gsem).wait()
            pltpu.async_copy(yp.at[ev.at[2 * j + 2]], rows1, gsem)
            pltpu.sync_copy(rows0, acc.at[ev.at[2 * j + 1]], add=True)
            pltpu.make_async_copy(yp.at[ev.at[2 * j + 2]], rows1, gsem).wait()

            @pl.when(j + 2 < CHP)
            def _():
                pltpu.async_copy(yp.at[ev.at[2 * j + 4]], rows0, gsem)

            pltpu.sync_copy(rows1, acc.at[ev.at[2 * j + 3]], add=True)
            return 0

        lax.fori_loop(0, CHP // 2, body, 0)
        return 0

    lax.fori_loop(0, PASSES, passloop, 0)
    plsc.subcore_barrier()
    pltpu.sync_copy(acc.at[rs], outp.at[cid, rs])


def _scat_call(yp, eb):
    return pl.kernel(
        _scat_body,
        out_type=jax.ShapeDtypeStruct((NC, NP, D), jnp.float32),
        mesh=_mesh(),
        scratch_types=[
            pltpu.VMEM((2 * CHP, B), jnp.int32),
            pltpu.VMEM((B, D), jnp.float32),
            pltpu.VMEM((B, D), jnp.float32),
            pltpu.VMEM_SHARED((NP, D), jnp.float32),
            pltpu.SemaphoreType.DMA,
        ],
    )(yp, eb)


def _t0_body(degp_ref, dinvp_ref):
    dinvp_ref[...] = lax.rsqrt(degp_ref[0] + degp_ref[1] + 1.0)


def _t1_body(x_ref, w_ref, dinv_ref, y_ref):
    xw = jnp.dot(x_ref[...], w_ref[...],
                 preferred_element_type=jnp.float32,
                 precision=lax.Precision.HIGHEST)
    y_ref[...] = xw * dinv_ref[...]


def _mix(z, g_ref):
    ga = g_ref[0]
    gb = g_ref[1]
    gc = g_ref[2]
    gd = g_ref[3]
    gr = jnp.where(z < 0, ga * z, z)
    gr = jnp.where((z >= 0) & (z < gc), gb * z, gr)
    gr = jnp.where(z >= gc, gd * z, gr)
    return _BETA * z + (_CMIX - _BETA) * gr


def _t2_body(p_ref, dinv_ref, b_ref, g_ref, w_ref, y2_ref):
    dcol = dinv_ref[...]
    z = dcol * (p_ref[0] + p_ref[1]) + b_ref[...][None, :]
    h = _mix(z, g_ref)
    xw2 = jnp.dot(h, w_ref[...],
                  preferred_element_type=jnp.float32,
                  precision=lax.Precision.HIGHEST)
    y2_ref[...] = xw2 * dcol


def _t3_body(p_ref, dinv_ref, b_ref, g_ref, o_ref):
    z = dinv_ref[...] * (p_ref[0] + p_ref[1]) + b_ref[...][None, :]
    o_ref[...] = _mix(z, g_ref)


_RB = 1024
_GRID = NP // _RB


def _row_specs():
    sp_p = pl.BlockSpec((NC, _RB, D), lambda i: (0, i, 0))
    sp_r = pl.BlockSpec((_RB, D), lambda i: (i, 0))
    sp_c = pl.BlockSpec((_RB, 1), lambda i: (i, 0))
    sp_b = pl.BlockSpec((D,), lambda i: (0,))
    sp_g = pl.BlockSpec((4,), lambda i: (0,))
    sp_w = pl.BlockSpec((D, D), lambda i: (0, 0))
    return sp_p, sp_r, sp_c, sp_b, sp_g, sp_w


def kernel(x, edge_index, W1, b1, g1, W2, b2, g2):
    real = edge_index.reshape(2, E // B, B).transpose(1, 0, 2).reshape(2 * E // B, B)
    npad = EP - E
    padblk = (N + (jnp.arange(2 * npad, dtype=jnp.int32) % (NP - N))
              ).reshape(2 * npad // B, B)
    eb = jnp.concatenate([real, padblk], axis=0)

    xp = jnp.pad(x, ((0, NP - N), (0, 0)))

    degp = _deg_call(eb).reshape(NC, NP // D, D)

    dinvp = pl.pallas_call(
        _t0_body,
        out_shape=jax.ShapeDtypeStruct((NP // D, D), jnp.float32),
    )(degp)
    dinv = dinvp.reshape(NP, 1)

    sp_p, sp_r, sp_c, sp_b, sp_g, sp_w = _row_specs()
    y1 = pl.pallas_call(
        _t1_body,
        grid=(_GRID,),
        in_specs=[sp_r, sp_w, sp_c],
        out_specs=sp_r,
        out_shape=jax.ShapeDtypeStruct((NP, D), jnp.float32),
    )(xp, W1, dinv)

    p1 = _scat_call(y1, eb)

    y2 = pl.pallas_call(
        _t2_body,
        grid=(_GRID,),
        in_specs=[sp_p, sp_c, sp_b, sp_g, sp_w],
        out_specs=sp_r,
        out_shape=jax.ShapeDtypeStruct((NP, D), jnp.float32),
    )(p1, dinv, b1, g1, W2)

    p2 = _scat_call(y2, eb)

    return pl.pallas_call(
        _t3_body,
        grid=(_GRID,),
        in_specs=[sp_p, sp_c, sp_b, sp_g],
        out_specs=sp_r,
        out_shape=jax.ShapeDtypeStruct((N, D), jnp.float32),
    )(p2, dinv, b2, g2)
